# baseline (speedup 1.0000x reference)
import functools

import jax
import jax.numpy as jnp
from jax import lax
from jax.experimental import pallas as pl
from jax.experimental.pallas import tpu as pltpu
from jax.experimental.pallas import tpu_sc as plsc

L = 16
NW = 32
B, S = 4, 2048
TOK = B * S
TPW = TOK // NW
HID = 128
NCH = HID // L
CPB = S // TPW


def _body(ids_hbm, tt_hbm, word_hbm, pos_hbm, type_hbm, gamma_hbm, beta_hbm,
          out_hbm, idx_v, tti_v, w_v, p_v, t_v, g_v, b_v, sem):
    c = lax.axis_index("c")
    s = lax.axis_index("s")
    wid = s * 2 + c
    base = wid * TPW

    pltpu.sync_copy(ids_hbm.at[pl.ds(wid * 2, 2)], idx_v)
    pltpu.sync_copy(tt_hbm.at[pl.ds(wid * 2, 2)], tti_v)

    cps = [
        pltpu.async_copy(word_hbm.at[idx_v.at[0]], w_v.at[pl.ds(0, 128)], sem),
        pltpu.async_copy(word_hbm.at[idx_v.at[1]], w_v.at[pl.ds(128, 128)], sem),
        pltpu.async_copy(type_hbm.at[tti_v.at[0]], t_v.at[pl.ds(0, 128)], sem),
        pltpu.async_copy(type_hbm.at[tti_v.at[1]], t_v.at[pl.ds(128, 128)], sem),
    ]

    pos_start = (wid % CPB) * TPW
    pltpu.sync_copy(pos_hbm.at[pl.ds(pos_start, TPW)], p_v)
    pltpu.sync_copy(gamma_hbm, g_v)
    pltpu.sync_copy(beta_hbm, b_v)
    for cp in cps:
        cp.wait()

    inv_hid = 1.0 / HID

    def lane_sum(x):
        idx = lax.iota(jnp.int32, L)
        dnums = lax.GatherDimensionNumbers(
            offset_dims=(), collapsed_slice_dims=(0,), start_index_map=(0,))
        for k in (1, 2, 4, 8):
            perm = (idx ^ k).reshape(L, 1)
            x = x + lax.gather(x, perm, dnums, slice_sizes=(1,),
                               mode=lax.GatherScatterMode.PROMISE_IN_BOUNDS)
        return x

    def token(i, carry):
        sls = [pl.ds(j * L, L) for j in range(NCH)]
        e = [w_v[i, sl] + p_v[i, sl] + t_v[i, sl] for sl in sls]
        tot = e[0]
        for j in range(1, NCH):
            tot = tot + e[j]
        mean = lane_sum(tot) * inv_hid
        d = [ej - mean for ej in e]
        sq = d[0] * d[0]
        for j in range(1, NCH):
            sq = sq + d[j] * d[j]
        vv = lane_sum(sq) * inv_hid + 1e-12
        bits = lax.bitcast_convert_type(vv, jnp.int32)
        y = lax.bitcast_convert_type(jnp.int32(0x5F3759DF) - (bits >> 1),
                                     jnp.float32)
        half = vv * 0.5
        y = y * (1.5 - half * y * y)
        y = y * (1.5 - half * y * y)
        y = y * (1.5 - half * y * y)
        for j in range(NCH):
            w_v[i, sls[j]] = d[j] * y * g_v[sls[j]] + b_v[sls[j]]
        return carry

    lax.fori_loop(0, TPW, token, 0)

    pltpu.sync_copy(w_v, out_hbm.at[pl.ds(base, TPW)])


def kernel(input_ids, token_type_ids, word_table, pos_table, type_table,
           gamma, beta):
    ids = input_ids.reshape(TOK // 128, 128).astype(jnp.int32)
    tts = token_type_ids.reshape(TOK // 128, 128).astype(jnp.int32)
    mesh = plsc.VectorSubcoreMesh(core_axis_name="c", subcore_axis_name="s")
    run = pl.kernel(
        _body,
        out_type=jax.ShapeDtypeStruct((TOK, HID), jnp.float32),
        mesh=mesh,
        scratch_types=[
            pltpu.VMEM((2, 128), jnp.int32),
            pltpu.VMEM((2, 128), jnp.int32),
            pltpu.VMEM((TPW, HID), jnp.float32),
            pltpu.VMEM((TPW, HID), jnp.float32),
            pltpu.VMEM((TPW, HID), jnp.float32),
            pltpu.VMEM((HID,), jnp.float32),
            pltpu.VMEM((HID,), jnp.float32),
            pltpu.SemaphoreType.DMA,
        ],
    )
    out = run(ids, tts, word_table, pos_table, type_table, gamma, beta)
    return out.reshape(B, S, HID)
